# baseline (speedup 1.0000x reference)
import functools

import jax
import jax.numpy as jnp
from jax import lax
from jax.experimental import pallas as pl
from jax.experimental.pallas import tpu as pltpu
from jax.experimental.pallas import tpu_sc as plsc

VOCAB = 100000
DIM = 64
BATCH = 4096
SEQ = 200
NC = 2
NS = 16
NW = NC * NS
ROWS_PER_W = BATCH // NW
NB = 1
CHUNK = NB * SEQ
NBUF = 4
NCHUNK = ROWS_PER_W // NB
NROUND = NCHUNK // NBUF


@functools.partial(
    pl.kernel,
    mesh=plsc.VectorSubcoreMesh(core_axis_name="c", subcore_axis_name="s"),
    out_type=jax.ShapeDtypeStruct((BATCH, SEQ, DIM), jnp.float32),
    scratch_types=(
        [pltpu.VMEM((CHUNK,), jnp.int32) for _ in range(NBUF)]
        + [pltpu.VMEM((CHUNK, DIM), jnp.float32) for _ in range(NBUF)]
        + [pltpu.SemaphoreType.DMA for _ in range(2 * NBUF)]
    ),
    compiler_params=pltpu.CompilerParams(use_tc_tiling_on_sc=False),
)
def _gather_kernel(idx_hbm, table_hbm, out_hbm, *scratch):
    idx = scratch[:NBUF]
    rows = scratch[NBUF:2 * NBUF]
    sg = scratch[2 * NBUF:3 * NBUF]
    ss = scratch[3 * NBUF:4 * NBUF]

    wid = lax.axis_index("s") * NC + lax.axis_index("c")
    base = wid * ROWS_PER_W

    def start_chunk(b, brow):
        pltpu.sync_copy(idx_hbm.at[pl.ds(brow * SEQ, CHUNK)], idx[b])
        pltpu.async_copy(table_hbm.at[idx[b]], rows[b], sg[b])

    for b in range(NBUF):
        start_chunk(b, base + b * NB)

    def body(r, _):
        brow0 = base + r * NBUF * NB
        for b in range(NBUF):
            brow = brow0 + b * NB
            pltpu.make_async_copy(table_hbm.at[idx[b]], rows[b], sg[b]).wait()
            pltpu.async_copy(rows[b], out_hbm.at[brow], ss[b])

            @pl.when(r < NROUND - 1)
            def _():
                pltpu.make_async_copy(
                    rows[b], out_hbm.at[brow], ss[b]).wait()
                start_chunk(b, brow + NBUF * NB)
        return ()

    lax.fori_loop(0, NROUND, body, ())

    last = base + (NROUND - 1) * NBUF * NB
    for b in range(NBUF):
        pltpu.make_async_copy(
            rows[b], out_hbm.at[last + b * NB], ss[b]).wait()


def kernel(x, table):
    return _gather_kernel(x.reshape(BATCH * SEQ), table)
